# baseline (speedup 1.0000x reference)
import functools

import jax
import jax.numpy as jnp
from jax import lax
from jax.experimental import pallas as pl
from jax.experimental.pallas import tpu as pltpu
from jax.experimental.pallas import tpu_sc as plsc

B = 16384
F = 26
K = 16
L = 16
NC = 2
NS = 16
NW = NC * NS
BPW = B // NW
C = 128
NCHUNK = BPW // C
G = C * F


def _fm_body(x_hbm, emb_hbm, fc_hbm, lw_hbm, lb_hbm, out_hbm,
             idx_v, fc_v, rows_v, u_v, out_v, lw_v, lb_v, sem_e, sem_f):
    wid = lax.axis_index("s") * NC + lax.axis_index("c")
    ibase = wid * (BPW * F)

    pltpu.sync_copy(x_hbm.at[pl.ds(ibase, BPW * F)], idx_v)
    pltpu.sync_copy(lw_hbm, lw_v)
    pltpu.sync_copy(lb_hbm, lb_v)
    fc_dma = pltpu.async_copy(fc_hbm.at[idx_v], fc_v, sem_f)

    iota = lax.iota(jnp.int32, L)

    def chunk_body(c, carry):
        idx_slice = idx_v.at[pl.ds(c * G, G)]
        pltpu.async_copy(emb_hbm.at[idx_slice], rows_v, sem_e).wait()

        def batch_body(b, carry2):
            off = b * F
            acc = rows_v[off]
            acc2 = acc * acc
            for f in range(1, F):
                v = rows_v[off + f]
                acc = acc + v
                acc2 = acc2 + v * v
            u_v[pl.ds((c * C + b) * K, K)] = acc * acc - acc2
            return carry2

        lax.fori_loop(0, C, batch_body, 0)
        return carry

    lax.fori_loop(0, NCHUNK, chunk_body, 0)
    fc_dma.wait()

    lw = lw_v[...]
    lb = lb_v[...]

    def group_body(g, carry):
        rowids = g * L + iota
        t = plsc.load_gather(u_v, [rowids * K])
        for k in range(1, K):
            t = t + plsc.load_gather(u_v, [rowids * K + k])
        fs = plsc.load_gather(fc_v, [rowids * F])
        for f in range(1, F):
            fs = fs + plsc.load_gather(fc_v, [rowids * F + f])
        z = lw * fs + lb + 0.5 * t
        out_v[pl.ds(g * L, L)] = 1.0 / (1.0 + jnp.exp(-z))
        return carry

    lax.fori_loop(0, BPW // L, group_body, 0)
    pltpu.sync_copy(out_v, out_hbm.at[pl.ds(wid * BPW, BPW)])


_fm_kernel = functools.partial(
    pl.kernel,
    out_type=jax.ShapeDtypeStruct((B,), jnp.float32),
    mesh=plsc.VectorSubcoreMesh(core_axis_name="c", subcore_axis_name="s"),
    compiler_params=pltpu.CompilerParams(
        needs_layout_passes=False, use_tc_tiling_on_sc=False),
    scratch_types=[
        pltpu.VMEM((BPW * F,), jnp.int32),
        pltpu.VMEM((BPW * F,), jnp.float32),
        pltpu.VMEM((G, K), jnp.float32),
        pltpu.VMEM((BPW * K,), jnp.float32),
        pltpu.VMEM((BPW,), jnp.float32),
        pltpu.VMEM((L,), jnp.float32),
        pltpu.VMEM((L,), jnp.float32),
        pltpu.SemaphoreType.DMA,
        pltpu.SemaphoreType.DMA,
    ],
)(_fm_body)


def kernel(x, emb_table, fc_table, lin_w, lin_b, W0, b0, W1, b1, W2, b2):
    xf = x.reshape(B * F)
    fcf = fc_table.reshape(-1)
    lw16 = jnp.broadcast_to(lin_w.reshape(1), (L,))
    lb16 = jnp.broadcast_to(lin_b.reshape(1), (L,))
    return _fm_kernel(xf, emb_table, fcf, lw16, lb16)
